# baseline (speedup 1.0000x reference)
import jax
import jax.numpy as jnp
from jax.experimental import pallas as pl
from jax.experimental.pallas import tpu as pltpu

_CHUNKS = (256, 512, 1024, 2304)
_OFFS = (0, 256, 768, 1792)
_N = len(_CHUNKS)


def _copy_kernel(x_ref, o_ref, buf, sem_in, sem_out):
    ins = [
        pltpu.make_async_copy(
            x_ref.at[pl.ds(_OFFS[i], _CHUNKS[i])],
            buf.at[pl.ds(_OFFS[i], _CHUNKS[i])],
            sem_in.at[i],
        )
        for i in range(_N)
    ]
    outs = [
        pltpu.make_async_copy(
            buf.at[pl.ds(_OFFS[i], _CHUNKS[i])],
            o_ref.at[pl.ds(_OFFS[i], _CHUNKS[i])],
            sem_out.at[i],
        )
        for i in range(_N)
    ]
    for c in ins:
        c.start()
    for i in range(_N):
        ins[i].wait()
        outs[i].start()
    for c in outs:
        c.wait()


def kernel(x, embed):
    del embed
    rows, cols = x.shape
    return pl.pallas_call(
        _copy_kernel,
        out_shape=jax.ShapeDtypeStruct(x.shape, x.dtype),
        in_specs=[pl.BlockSpec(memory_space=pl.ANY)],
        out_specs=pl.BlockSpec(memory_space=pl.ANY),
        scratch_shapes=[
            pltpu.VMEM((rows, cols), x.dtype),
            pltpu.SemaphoreType.DMA((_N,)),
            pltpu.SemaphoreType.DMA((_N,)),
        ],
    )(x)


# speedup vs baseline: 1.0956x; 1.0956x over previous
import jax
import jax.numpy as jnp
from jax.experimental import pallas as pl
from jax.experimental.pallas import tpu as pltpu

_CHUNKS = (512, 512, 512, 512, 512, 512, 512, 512)
_OFFS = (0, 512, 1024, 1536, 2048, 2560, 3072, 3584)
_N = len(_CHUNKS)


def _copy_kernel(x_ref, o_ref, buf, sem_in, sem_out):
    ins = [
        pltpu.make_async_copy(
            x_ref.at[pl.ds(_OFFS[i], _CHUNKS[i])],
            buf.at[pl.ds(_OFFS[i], _CHUNKS[i])],
            sem_in.at[i],
        )
        for i in range(_N)
    ]
    outs = [
        pltpu.make_async_copy(
            buf.at[pl.ds(_OFFS[i], _CHUNKS[i])],
            o_ref.at[pl.ds(_OFFS[i], _CHUNKS[i])],
            sem_out.at[i],
        )
        for i in range(_N)
    ]
    for c in ins:
        c.start()
    for i in range(_N):
        ins[i].wait()
        outs[i].start()
    for c in outs:
        c.wait()


def kernel(x, embed):
    del embed
    rows, cols = x.shape
    return pl.pallas_call(
        _copy_kernel,
        out_shape=jax.ShapeDtypeStruct(x.shape, x.dtype),
        in_specs=[pl.BlockSpec(memory_space=pl.ANY)],
        out_specs=pl.BlockSpec(memory_space=pl.ANY),
        scratch_shapes=[
            pltpu.VMEM((rows, cols), x.dtype),
            pltpu.SemaphoreType.DMA((_N,)),
            pltpu.SemaphoreType.DMA((_N,)),
        ],
    )(x)


# speedup vs baseline: 1.1076x; 1.0110x over previous
import jax
import jax.numpy as jnp
from jax.experimental import pallas as pl
from jax.experimental.pallas import tpu as pltpu

_CHUNKS = (1024, 1024, 1024, 1024)
_OFFS = (0, 1024, 2048, 3072)
_N = len(_CHUNKS)


def _copy_kernel(x_ref, o_ref, buf, sem_in, sem_out):
    ins = [
        pltpu.make_async_copy(
            x_ref.at[pl.ds(_OFFS[i], _CHUNKS[i])],
            buf.at[pl.ds(_OFFS[i], _CHUNKS[i])],
            sem_in.at[i],
        )
        for i in range(_N)
    ]
    outs = [
        pltpu.make_async_copy(
            buf.at[pl.ds(_OFFS[i], _CHUNKS[i])],
            o_ref.at[pl.ds(_OFFS[i], _CHUNKS[i])],
            sem_out.at[i],
        )
        for i in range(_N)
    ]
    for c in ins:
        c.start()
    for i in range(_N):
        ins[i].wait()
        outs[i].start()
    for c in outs:
        c.wait()


def kernel(x, embed):
    del embed
    rows, cols = x.shape
    return pl.pallas_call(
        _copy_kernel,
        out_shape=jax.ShapeDtypeStruct(x.shape, x.dtype),
        in_specs=[pl.BlockSpec(memory_space=pl.ANY)],
        out_specs=pl.BlockSpec(memory_space=pl.ANY),
        scratch_shapes=[
            pltpu.VMEM((rows, cols), x.dtype),
            pltpu.SemaphoreType.DMA((_N,)),
            pltpu.SemaphoreType.DMA((_N,)),
        ],
    )(x)
